# baseline (speedup 1.0000x reference)
import functools

import jax
import jax.numpy as jnp
from jax import lax
from jax.experimental import pallas as pl
from jax.experimental.pallas import tpu as pltpu
from jax.experimental.pallas import tpu_sc as plsc

_L = 16


def _make_sc_gather_score(B, D, NNEG):
    NW = 32
    BW = B // NW
    CB = 64
    NCH = BW // CB
    NB = NNEG * CB
    SH = 2 * CB + NB
    HALF = D

    mesh = plsc.VectorSubcoreMesh(
        core_axis_name="c", subcore_axis_name="s", num_cores=2, num_subcores=16
    )

    @functools.partial(
        pl.kernel,
        out_type=(
            jax.ShapeDtypeStruct((B,), jnp.float32),
            jax.ShapeDtypeStruct((B,), jnp.float32),
        ),
        mesh=mesh,
        scratch_types=[
            pltpu.VMEM((BW,), jnp.int32),
            pltpu.VMEM((BW,), jnp.int32),
            pltpu.VMEM((BW * NNEG,), jnp.int32),
            pltpu.VMEM((2 * SH,), jnp.int32),
            pltpu.VMEM((2, CB, 2 * D), jnp.float32),
            pltpu.VMEM((2, CB, 2 * D), jnp.float32),
            pltpu.VMEM((2, NB, 2 * D), jnp.float32),
            pltpu.VMEM((2 * CB,), jnp.float32),
            pltpu.VMEM((2 * CB,), jnp.float32),
            pltpu.SemaphoreType.DMA,
            pltpu.SemaphoreType.DMA,
        ],
        compiler_params=pltpu.CompilerParams(needs_layout_passes=False),
    )
    def sc_fn(u_pos_h, v_pos_h, vneg_h, u_tab, v_tab, pos_out, neg_out,
              u_idx, v_idx, n_idx, sh, u_rows, v_rows, n_rows, pos_b, neg_b,
              sem0, sem1):
        wid = lax.axis_index("c") * 16 + lax.axis_index("s")
        base = wid * BW
        pltpu.sync_copy(u_pos_h.at[pl.ds(base, BW)], u_idx)
        pltpu.sync_copy(v_pos_h.at[pl.ds(base, BW)], v_idx)
        pltpu.sync_copy(vneg_h.at[pl.ds(base * NNEG, BW * NNEG)], n_idx)

        sems = (sem0, sem1)

        def shift_chunk(c):
            off = (c % 2) * SH

            def sbody(g, carry):
                sh[pl.ds(off + g * _L, _L)] = (
                    u_idx[pl.ds(c * CB + g * _L, _L)] >> 1)
                sh[pl.ds(off + CB + g * _L, _L)] = (
                    v_idx[pl.ds(c * CB + g * _L, _L)] >> 1)
                return carry

            lax.fori_loop(0, CB // _L, sbody, 0)

            def nbody(g, carry):
                sh[pl.ds(off + 2 * CB + g * _L, _L)] = (
                    n_idx[pl.ds(c * NB + g * _L, _L)] >> 1)
                return carry

            lax.fori_loop(0, NB // _L, nbody, 0)

        def fire(c):
            p = c % 2
            s = sems[p]
            off = p * SH
            hs = [
                pltpu.async_copy(
                    u_tab.at[sh.at[pl.ds(off, CB)]], u_rows.at[p], s),
                pltpu.async_copy(
                    v_tab.at[sh.at[pl.ds(off + CB, CB)]], v_rows.at[p], s),
            ]
            o = 0
            while o < NB:
                n = min(128, NB - o)
                hs.append(pltpu.async_copy(
                    v_tab.at[sh.at[pl.ds(off + 2 * CB + o, n)]],
                    n_rows.at[p, pl.ds(o, n)], s))
                o += n
            return hs

        def compute_chunk(c):
            p = c % 2
            ub = u_rows.at[p]
            vb = v_rows.at[p]
            nb = n_rows.at[p]

            def gbody(g, carry):
                iota = lax.iota(jnp.int32, _L)
                e0 = c * CB + g * _L
                row = g * _L + iota
                cu0 = (u_idx[pl.ds(e0, _L)] & 1) * HALF
                cv0 = (v_idx[pl.ds(e0, _L)] & 1) * HALF
                nrow = []
                cn0 = []
                for j in range(NNEG):
                    nid = plsc.load_gather(n_idx, [(e0 + iota) * NNEG + j])
                    cn0.append((nid & 1) * HALF)
                    nrow.append(row * NNEG + j)

                def dbody(d, accs):
                    ap, an = accs
                    gu = plsc.load_gather(ub, [row, cu0 + d])
                    gv = plsc.load_gather(vb, [row, cv0 + d])
                    ns = None
                    for j in range(NNEG):
                        gn = plsc.load_gather(nb, [nrow[j], cn0[j] + d])
                        ns = gn if ns is None else ns + gn
                    return (ap + gu * gv, an + gu * ns)

                zz = jnp.zeros((_L,), jnp.float32)
                ap, an = lax.fori_loop(0, D, dbody, (zz, zz))
                pos_b[pl.ds(p * CB + g * _L, _L)] = ap
                neg_b[pl.ds(p * CB + g * _L, _L)] = an
                return carry

            lax.fori_loop(0, CB // _L, gbody, 0)

        shift_chunk(0)
        pending = {0: fire(0)}
        for c in range(NCH):
            p = c % 2
            if c + 1 < NCH:
                shift_chunk(c + 1)
                pending[c + 1] = fire(c + 1)
            for h in pending.pop(c):
                h.wait()
            compute_chunk(c)
            pltpu.sync_copy(pos_b.at[pl.ds(p * CB, CB)],
                            pos_out.at[pl.ds(base + c * CB, CB)])
            pltpu.sync_copy(neg_b.at[pl.ds(p * CB, CB)],
                            neg_out.at[pl.ds(base + c * CB, CB)])

    return sc_fn


def _finish(pos, neg, B):
    def body(p_ref, n_ref, o_ref):
        p = p_ref[...]
        n = n_ref[...]
        tot = jax.nn.log_sigmoid(p) + jax.nn.log_sigmoid(-n)
        o_ref[0, 0] = -jnp.sum(tot) / B

    return pl.pallas_call(
        body,
        out_shape=jax.ShapeDtypeStruct((1, 1), jnp.float32),
        in_specs=[
            pl.BlockSpec(memory_space=pltpu.VMEM),
            pl.BlockSpec(memory_space=pltpu.VMEM),
        ],
        out_specs=pl.BlockSpec(memory_space=pltpu.SMEM),
    )(pos, neg)


def kernel(u_pos, v_pos, v_neg, batch_size, U, V):
    B = u_pos.shape[0]
    D = U.shape[1]
    NNEG = v_neg.shape[1]
    vneg_flat = v_neg.reshape(B * NNEG)
    u_wide = U.reshape(-1, 2 * D)
    v_wide = V.reshape(-1, 2 * D)
    sc_fn = _make_sc_gather_score(B, D, NNEG)
    pos, neg = sc_fn(u_pos, v_pos, vneg_flat, u_wide, v_wide)
    out = _finish(pos.reshape(128, -1), neg.reshape(128, -1), B)
    return out[0, 0]


# speedup vs baseline: 1.6015x; 1.6015x over previous
import functools

import jax
import jax.numpy as jnp
from jax import lax
from jax.experimental import pallas as pl
from jax.experimental.pallas import tpu as pltpu
from jax.experimental.pallas import tpu_sc as plsc

_L = 16


def _make_sc_gather_score(B, D, NNEG):
    NW = 32
    BW = B // NW
    CB = 64
    NCH = BW // CB
    KD = D // _L
    NG = CB // _L

    mesh = plsc.VectorSubcoreMesh(
        core_axis_name="c", subcore_axis_name="s", num_cores=2, num_subcores=16
    )

    @functools.partial(
        pl.kernel,
        out_type=jax.ShapeDtypeStruct((B, 2 * _L), jnp.float32),
        mesh=mesh,
        scratch_types=[
            pltpu.VMEM((CB,), jnp.int32),
            pltpu.VMEM((CB,), jnp.int32),
            pltpu.VMEM((CB * NNEG,), jnp.int32),
            pltpu.VMEM((2, CB, D), jnp.float32),
            pltpu.VMEM((2, CB, D), jnp.float32),
            pltpu.VMEM((2, NNEG, CB, D), jnp.float32),
            pltpu.VMEM((CB, 2 * _L), jnp.float32),
            pltpu.SemaphoreType.DMA,
            pltpu.SemaphoreType.DMA,
        ],
    )
    def sc_fn(u_pos_h, v_pos_h, vneg_h, u_tab, v_tab, out_h,
              u_idx, v_idx, n_idx, u_rows, v_rows, n_rows, pbuf, sem0, sem1):
        wid = lax.axis_index("c") * 16 + lax.axis_index("s")
        base = wid * BW

        sems = (sem0, sem1)

        def fire(c, p):
            s = sems[p]
            pltpu.sync_copy(u_pos_h.at[pl.ds(base + c * CB, CB)], u_idx)
            pltpu.sync_copy(v_pos_h.at[pl.ds(base + c * CB, CB)], v_idx)
            pltpu.sync_copy(
                vneg_h.at[pl.ds((base + c * CB) * NNEG, CB * NNEG)], n_idx)

            def issue(g, carry):
                uvec = u_idx[pl.ds(g * _L, _L)]
                vvec = v_idx[pl.ds(g * _L, _L)]
                nvec = [n_idx[pl.ds(g * _L * NNEG + q * _L, _L)]
                        for q in range(NNEG)]
                for t in range(_L):
                    i = g * _L + t
                    pltpu.async_copy(u_tab.at[uvec[t]], u_rows.at[p, i], s)
                    pltpu.async_copy(v_tab.at[vvec[t]], v_rows.at[p, i], s)
                    for j in range(NNEG):
                        q, r = divmod(t * NNEG + j, _L)
                        pltpu.async_copy(
                            v_tab.at[nvec[q][r]], n_rows.at[p, j, i], s)
                return carry

            lax.fori_loop(0, NG, issue, 0)

        def drain(p):
            s = sems[p]
            pltpu.make_async_copy(
                u_tab.at[pl.ds(0, CB)], u_rows.at[p], s).wait()
            pltpu.make_async_copy(
                u_tab.at[pl.ds(0, CB)], v_rows.at[p], s).wait()
            for j in range(NNEG):
                pltpu.make_async_copy(
                    u_tab.at[pl.ds(0, CB)], n_rows.at[p, j], s).wait()

        def compute_chunk(p):
            ub, vb, nb, pb = (u_rows.at[p], v_rows.at[p], n_rows.at[p],
                              pbuf)

            def body(i, carry):
                pos = None
                neg = None
                for k in range(KD):
                    sl = pl.ds(k * _L, _L)
                    u = ub[i, sl]
                    pp = u * vb[i, sl]
                    ns = nb[0, i, sl]
                    for j in range(1, NNEG):
                        ns = ns + nb[j, i, sl]
                    nn = u * ns
                    pos = pp if pos is None else pos + pp
                    neg = nn if neg is None else neg + nn
                pb[i, pl.ds(0, _L)] = pos
                pb[i, pl.ds(_L, _L)] = neg
                return carry

            lax.fori_loop(0, CB, body, 0, unroll=2)

        def emit(c, p):
            compute_chunk(p)
            pltpu.sync_copy(pbuf, out_h.at[pl.ds(base + c * CB, CB)])

        fire(0, 0)

        def pair(k, carry):
            c0 = 2 * k
            fire(c0 + 1, 1)
            drain(0)
            emit(c0, 0)

            @pl.when(k + 1 < NCH // 2)
            def _():
                fire(c0 + 2, 0)

            drain(1)
            emit(c0 + 1, 1)
            return carry

        lax.fori_loop(0, NCH // 2, pair, 0)

    return sc_fn


def _finish(part, B):
    def body(x_ref, o_ref):
        x = x_ref[...]
        pos = jnp.sum(x[:, :_L], axis=1)
        neg = jnp.sum(x[:, _L:], axis=1)
        tot = jax.nn.log_sigmoid(pos) + jax.nn.log_sigmoid(-neg)
        o_ref[0, 0] = -jnp.sum(tot) / B

    return pl.pallas_call(
        body,
        out_shape=jax.ShapeDtypeStruct((1, 1), jnp.float32),
        in_specs=[pl.BlockSpec(memory_space=pltpu.VMEM)],
        out_specs=pl.BlockSpec(memory_space=pltpu.SMEM),
    )(part)


def kernel(u_pos, v_pos, v_neg, batch_size, U, V):
    B = u_pos.shape[0]
    D = U.shape[1]
    NNEG = v_neg.shape[1]
    vneg_flat = v_neg.reshape(B * NNEG)
    sc_fn = _make_sc_gather_score(B, D, NNEG)
    part = sc_fn(u_pos, v_pos, vneg_flat, U, V)
    out = _finish(part, B)
    return out[0, 0]
